# baseline (speedup 1.0000x reference)
import functools

import jax
import jax.numpy as jnp
from jax import lax
from jax.experimental import pallas as pl
from jax.experimental.pallas import tpu as pltpu
from jax.experimental.pallas import tpu_sc as plsc

LANES = 16


def _build_sc_agg(n_nodes, d_feat, b, n_neigh, chunk, n_chunks_pt,
                  tile_rows, n_tiles, w_cols, inv_count):
    mesh = plsc.VectorSubcoreMesh(core_axis_name="c", subcore_axis_name="s")
    n_slots = n_neigh + 1
    ngr = n_neigh * chunk
    grows = ngr + chunk
    n_groups = w_cols // (2 * LANES)
    assert n_chunks_pt % 6 == 0
    cslab = n_nodes // n_tiles
    assert cslab * n_tiles == n_nodes
    cpiece = chunk
    n_pieces = -(-cslab // cpiece)
    n_pieces += n_pieces % 2

    @functools.partial(
        pl.kernel,
        mesh=mesh,
        out_type=jax.ShapeDtypeStruct((b, d_feat), jnp.float32),
        compiler_params=pltpu.CompilerParams(use_tc_tiling_on_sc=False,
                                             needs_layout_passes=False),
        scratch_types=[
            pltpu.VMEM_SHARED((n_nodes, w_cols), jnp.bfloat16),
            pltpu.VMEM((3, grows), jnp.int32),
            pltpu.VMEM((2, grows, w_cols), jnp.bfloat16),
            pltpu.VMEM((2, chunk, w_cols), jnp.float32),
            pltpu.SemaphoreType.DMA,
            pltpu.SemaphoreType.DMA,
            pltpu.SemaphoreType.DMA,
            pltpu.SemaphoreType.DMA,
            pltpu.SemaphoreType.DMA,
            pltpu.SemaphoreType.DMA,
            pltpu.SemaphoreType.DMA,
        ],
    )
    def agg(nodes_hbm, neighs_hbm, table_hbm, out_hbm,
            tblk, idx_v, buf_v, outb_v,
            gsem0, gsem1, osem0, osem1, isem0, isem1, isem2):
        cid = lax.axis_index("c")
        sid = lax.axis_index("s")
        gsems = (gsem0, gsem1)
        osems = (osem0, osem1)
        isems = (isem0, isem1, isem2)
        c0 = cid * w_cols

        def stage_row0(i):
            return jnp.minimum(sid * cslab + i * cpiece,
                               sid * cslab + cslab - cpiece)

        def stage_in(i, pp):
            return pltpu.make_async_copy(
                table_hbm.at[pl.ds(stage_row0(i), cpiece), pl.ds(c0, w_cols)],
                outb_v.at[pp], gsems[pp])

        def stage_out(i, pp):
            return pltpu.make_async_copy(
                buf_v.at[pp].at[pl.ds(0, cpiece)],
                tblk.at[pl.ds(stage_row0(i), cpiece)], osems[pp])

        stage_in(0, 0).start()

        def piece_body(j, _):
            for pp in (0, 1):
                i = 2 * j + pp
                ppn = (pp + 1) % 2

                @pl.when(i + 1 < n_pieces)
                def _():
                    stage_in(i + 1, ppn).start()

                stage_in(i, pp).wait()

                @pl.when(i >= 2)
                def _():
                    stage_out(i - 2, pp).wait()

                def crow(r4, _):
                    for dr in range(4):
                        r = r4 * 4 + dr
                        for h in range(n_groups):
                            a = outb_v[pp, r, pl.ds(h * 2 * LANES, LANES)]
                            bq = outb_v[pp, r,
                                        pl.ds(h * 2 * LANES + LANES, LANES)]
                            buf_v[pp, r, pl.ds(h * 2 * LANES, 2 * LANES)] = (
                                plsc.pack(a, bq,
                                          format=plsc.PackFormat.INTERLEAVED))
                    return 0

                lax.fori_loop(0, cpiece // 4, crow, 0)
                stage_out(i, pp).start()
            return 0

        lax.fori_loop(0, n_pieces // 2, piece_body, 0)
        for pp in (0, 1):
            stage_out(n_pieces - 2 + pp, pp).wait()
        plsc.subcore_barrier()

        def row_base(k):
            return jnp.minimum(sid * tile_rows + k * chunk, b - chunk)

        def idx_cps(k, q):
            rb = row_base(k)
            return (
                pltpu.make_async_copy(
                    neighs_hbm.at[pl.ds(rb * n_neigh, ngr)],
                    idx_v.at[q, pl.ds(0, ngr)], isems[q]),
                pltpu.make_async_copy(
                    nodes_hbm.at[pl.ds(rb, chunk)],
                    idx_v.at[q, pl.ds(ngr, chunk)], isems[q]),
            )

        def gather_cps(q, p, sem):
            return (
                pltpu.make_async_copy(
                    tblk.at[idx_v.at[q, pl.ds(0, ngr)]],
                    buf_v.at[p].at[pl.ds(0, ngr)], sem),
                pltpu.make_async_copy(
                    tblk.at[idx_v.at[q, pl.ds(ngr, chunk)]],
                    buf_v.at[p].at[pl.ds(ngr, chunk)], sem),
            )

        def start(cps):
            for cp in cps:
                cp.start()

        def wait(cps):
            for cp in cps:
                cp.wait()

        def out_slice(k):
            return out_hbm.at[pl.ds(row_base(k), chunk), pl.ds(c0, w_cols)]

        start(idx_cps(0, 0))
        wait(idx_cps(0, 0))
        start(gather_cps(0, 0, gsems[0]))
        start(idx_cps(1, 1))

        def six_body(i, _):
            for u in range(6):
                k = 6 * i + u
                p = u % 2
                pn = (p + 1) % 2
                q1 = (u + 1) % 3
                q2 = (u + 2) % 3

                @pl.when(k + 2 < n_chunks_pt)
                def _():
                    start(idx_cps(k + 2, q2))

                @pl.when(k + 1 < n_chunks_pt)
                def _():
                    wait(idx_cps(k + 1, q1))
                    start(gather_cps(q1, pn, gsems[pn]))

                wait(gather_cps(u % 3, p, gsems[p]))

                @pl.when(k >= 2)
                def _():
                    pltpu.make_async_copy(
                        outb_v.at[p], out_slice(k - 2), osems[p]).wait()

                def rbody(r4, _):
                    for dr in range(4):
                        r = r4 * 4 + dr
                        rn = r * n_neigh
                        for h in range(n_groups):
                            col = pl.ds(h * 2 * LANES, 2 * LANES)
                            acc = buf_v[p, ngr + r, col]
                            for s in range(n_neigh):
                                acc = acc + buf_v[p, rn + s, col]
                            lo, hi = plsc.unpack(
                                acc, format=plsc.PackFormat.INTERLEAVED)
                            outb_v[p, r, pl.ds(h * 2 * LANES, LANES)] = (
                                lo * inv_count)
                            outb_v[p, r, pl.ds(h * 2 * LANES + LANES,
                                               LANES)] = hi * inv_count
                    return 0

                lax.fori_loop(0, chunk // 4, rbody, 0)
                pltpu.async_copy(outb_v.at[p], out_slice(k), osems[p])
            return 0

        lax.fori_loop(0, n_chunks_pt // 6, six_body, 0)
        for p in (0, 1):
            k = n_chunks_pt - 2 + p
            pltpu.make_async_copy(
                outb_v.at[p], out_slice(k), osems[p]).wait()

    return agg


def kernel(nodes, to_neighs, feature_table, num_sample):
    b = nodes.shape[0]
    n_neigh = to_neighs.shape[1]
    n_nodes, d_feat = feature_table.shape
    n_tiles = 16
    chunk = 32
    w_cols = d_feat // 2
    inv_count = 1.0 / float(n_neigh + 1)

    tile_rows = -(-b // n_tiles)
    tile_rows += (-tile_rows) % 8
    n_chunks_pt = -(-tile_rows // chunk)
    n_chunks_pt += (-n_chunks_pt) % 6

    agg = _build_sc_agg(n_nodes, d_feat, b, n_neigh, chunk, n_chunks_pt,
                        tile_rows, n_tiles, w_cols, inv_count)
    return agg(nodes, to_neighs.reshape(-1), feature_table)


# speedup vs baseline: 1.0584x; 1.0584x over previous
import functools

import jax
import jax.numpy as jnp
from jax import lax
from jax.experimental import pallas as pl
from jax.experimental.pallas import tpu as pltpu
from jax.experimental.pallas import tpu_sc as plsc

LANES = 16


def _build_sc_agg(n_nodes, d_feat, b, n_neigh, chunk, n_chunks_pt,
                  tile_rows, n_tiles, w_cols, inv_count):
    mesh = plsc.VectorSubcoreMesh(core_axis_name="c", subcore_axis_name="s")
    n_slots = n_neigh + 1
    ngr = n_neigh * chunk
    grows = ngr + chunk
    n_groups = w_cols // (2 * LANES)
    assert n_chunks_pt % 6 == 0
    cslab = n_nodes // n_tiles
    assert cslab * n_tiles == n_nodes
    cpiece = chunk
    n_pieces = -(-cslab // cpiece)
    n_pieces += n_pieces % 2

    @functools.partial(
        pl.kernel,
        mesh=mesh,
        out_type=jax.ShapeDtypeStruct((b, d_feat), jnp.float32),
        compiler_params=pltpu.CompilerParams(use_tc_tiling_on_sc=False,
                                             needs_layout_passes=False),
        scratch_types=[
            pltpu.VMEM_SHARED((n_nodes, w_cols), jnp.bfloat16),
            pltpu.VMEM((3, grows), jnp.int32),
            pltpu.VMEM((2, grows, w_cols), jnp.bfloat16),
            pltpu.VMEM((2, chunk, w_cols), jnp.float32),
            pltpu.SemaphoreType.DMA,
            pltpu.SemaphoreType.DMA,
            pltpu.SemaphoreType.DMA,
            pltpu.SemaphoreType.DMA,
            pltpu.SemaphoreType.DMA,
            pltpu.SemaphoreType.DMA,
            pltpu.SemaphoreType.DMA,
        ],
    )
    def agg(nodes_hbm, neighs_hbm, table_hbm, out_hbm,
            tblk, idx_v, buf_v, outb_v,
            gsem0, gsem1, osem0, osem1, isem0, isem1, isem2):
        cid = lax.axis_index("c")
        sid = lax.axis_index("s")
        gsems = (gsem0, gsem1)
        osems = (osem0, osem1)
        isems = (isem0, isem1, isem2)
        c0 = cid * w_cols

        def stage_row0(i):
            return jnp.minimum(sid * cslab + i * cpiece,
                               sid * cslab + cslab - cpiece)

        def stage_in(i, pp):
            return pltpu.make_async_copy(
                table_hbm.at[pl.ds(stage_row0(i), cpiece), pl.ds(c0, w_cols)],
                outb_v.at[pp], gsems[pp])

        def stage_out(i, pp):
            return pltpu.make_async_copy(
                buf_v.at[pp].at[pl.ds(0, cpiece)],
                tblk.at[pl.ds(stage_row0(i), cpiece)], osems[pp])

        stage_in(0, 0).start()

        def piece_body(j, _):
            for pp in (0, 1):
                i = 2 * j + pp
                ppn = (pp + 1) % 2

                @pl.when(i + 1 < n_pieces)
                def _():
                    stage_in(i + 1, ppn).start()

                stage_in(i, pp).wait()

                @pl.when(i >= 2)
                def _():
                    stage_out(i - 2, pp).wait()

                def crow(r, _):
                    for h in range(n_groups):
                        a = outb_v[pp, r, pl.ds(h * 2 * LANES, LANES)]
                        bq = outb_v[pp, r, pl.ds(h * 2 * LANES + LANES, LANES)]
                        buf_v[pp, r, pl.ds(h * 2 * LANES, 2 * LANES)] = (
                            plsc.pack(a, bq,
                                      format=plsc.PackFormat.INTERLEAVED))
                    return 0

                lax.fori_loop(0, cpiece, crow, 0)
                stage_out(i, pp).start()
            return 0

        lax.fori_loop(0, n_pieces // 2, piece_body, 0)
        for pp in (0, 1):
            stage_out(n_pieces - 2 + pp, pp).wait()
        plsc.subcore_barrier()

        def row_base(k):
            return jnp.minimum(sid * tile_rows + k * chunk, b - chunk)

        def idx_cps(k, q):
            rb = row_base(k)
            return (
                pltpu.make_async_copy(
                    neighs_hbm.at[pl.ds(rb * n_neigh, ngr)],
                    idx_v.at[q, pl.ds(0, ngr)], isems[q]),
                pltpu.make_async_copy(
                    nodes_hbm.at[pl.ds(rb, chunk)],
                    idx_v.at[q, pl.ds(ngr, chunk)], isems[q]),
            )

        def gather_cps(q, p, sem):
            return (
                pltpu.make_async_copy(
                    tblk.at[idx_v.at[q, pl.ds(0, ngr)]],
                    buf_v.at[p].at[pl.ds(0, ngr)], sem),
                pltpu.make_async_copy(
                    tblk.at[idx_v.at[q, pl.ds(ngr, chunk)]],
                    buf_v.at[p].at[pl.ds(ngr, chunk)], sem),
            )

        def start(cps):
            for cp in cps:
                cp.start()

        def wait(cps):
            for cp in cps:
                cp.wait()

        def out_slice(k):
            return out_hbm.at[pl.ds(row_base(k), chunk), pl.ds(c0, w_cols)]

        start(idx_cps(0, 0))
        wait(idx_cps(0, 0))
        start(gather_cps(0, 0, gsems[0]))
        start(idx_cps(1, 1))

        def six_body(i, _):
            for u in range(6):
                k = 6 * i + u
                p = u % 2
                pn = (p + 1) % 2
                q1 = (u + 1) % 3
                q2 = (u + 2) % 3

                @pl.when(k + 2 < n_chunks_pt)
                def _():
                    start(idx_cps(k + 2, q2))

                @pl.when(k + 1 < n_chunks_pt)
                def _():
                    wait(idx_cps(k + 1, q1))
                    start(gather_cps(q1, pn, gsems[pn]))

                wait(gather_cps(u % 3, p, gsems[p]))

                @pl.when(k >= 2)
                def _():
                    pltpu.make_async_copy(
                        outb_v.at[p], out_slice(k - 2), osems[p]).wait()

                def rbody(r2, _):
                    for dr in range(2):
                        r = r2 * 2 + dr
                        rn = r * n_neigh
                        for h in range(n_groups):
                            col = pl.ds(h * 2 * LANES, 2 * LANES)
                            acc = buf_v[p, ngr + r, col]
                            for s in range(n_neigh):
                                acc = acc + buf_v[p, rn + s, col]
                            lo, hi = plsc.unpack(
                                acc, format=plsc.PackFormat.INTERLEAVED)
                            outb_v[p, r, pl.ds(h * 2 * LANES, LANES)] = (
                                lo * inv_count)
                            outb_v[p, r, pl.ds(h * 2 * LANES + LANES,
                                               LANES)] = hi * inv_count
                    return 0

                lax.fori_loop(0, chunk // 2, rbody, 0)
                pltpu.async_copy(outb_v.at[p], out_slice(k), osems[p])
            return 0

        lax.fori_loop(0, n_chunks_pt // 6, six_body, 0)
        for p in (0, 1):
            k = n_chunks_pt - 2 + p
            pltpu.make_async_copy(
                outb_v.at[p], out_slice(k), osems[p]).wait()

    return agg


def kernel(nodes, to_neighs, feature_table, num_sample):
    b = nodes.shape[0]
    n_neigh = to_neighs.shape[1]
    n_nodes, d_feat = feature_table.shape
    n_tiles = 16
    chunk = 32
    w_cols = d_feat // 2
    inv_count = 1.0 / float(n_neigh + 1)

    tile_rows = -(-b // n_tiles)
    tile_rows += (-tile_rows) % 8
    n_chunks_pt = -(-tile_rows // chunk)
    n_chunks_pt += (-n_chunks_pt) % 6

    agg = _build_sc_agg(n_nodes, d_feat, b, n_neigh, chunk, n_chunks_pt,
                        tile_rows, n_tiles, w_cols, inv_count)
    return agg(nodes, to_neighs.reshape(-1), feature_table)


# speedup vs baseline: 1.1125x; 1.0512x over previous
import functools

import jax
import jax.numpy as jnp
from jax import lax
from jax.experimental import pallas as pl
from jax.experimental.pallas import tpu as pltpu
from jax.experimental.pallas import tpu_sc as plsc

LANES = 16


def _build_sc_agg(n_nodes, d_feat, b, n_neigh, chunk, n_chunks_pt,
                  tile_rows, n_tiles, w_cols, inv_count):
    mesh = plsc.VectorSubcoreMesh(core_axis_name="c", subcore_axis_name="s")
    n_slots = n_neigh + 1
    ngr = n_neigh * chunk
    grows = ngr + chunk
    n_groups = w_cols // (2 * LANES)
    assert n_chunks_pt % 6 == 0
    cslab = n_nodes // n_tiles
    assert cslab * n_tiles == n_nodes
    cpiece = 56
    n_pieces = -(-cslab // cpiece)
    n_pieces += n_pieces % 2

    @functools.partial(
        pl.kernel,
        mesh=mesh,
        out_type=jax.ShapeDtypeStruct((b, d_feat), jnp.float32),
        compiler_params=pltpu.CompilerParams(use_tc_tiling_on_sc=False,
                                             needs_layout_passes=False),
        scratch_types=[
            pltpu.VMEM_SHARED((n_nodes, w_cols), jnp.bfloat16),
            pltpu.VMEM((3, grows), jnp.int32),
            pltpu.VMEM((2, grows, w_cols), jnp.bfloat16),
            pltpu.VMEM((2, cpiece, w_cols), jnp.float32),
            pltpu.SemaphoreType.DMA,
            pltpu.SemaphoreType.DMA,
            pltpu.SemaphoreType.DMA,
            pltpu.SemaphoreType.DMA,
            pltpu.SemaphoreType.DMA,
            pltpu.SemaphoreType.DMA,
            pltpu.SemaphoreType.DMA,
        ],
    )
    def agg(nodes_hbm, neighs_hbm, table_hbm, out_hbm,
            tblk, idx_v, buf_v, outb_v,
            gsem0, gsem1, osem0, osem1, isem0, isem1, isem2):
        cid = lax.axis_index("c")
        sid = lax.axis_index("s")
        gsems = (gsem0, gsem1)
        osems = (osem0, osem1)
        isems = (isem0, isem1, isem2)
        c0 = cid * w_cols

        def stage_row0(i):
            return jnp.minimum(sid * cslab + i * cpiece,
                               sid * cslab + cslab - cpiece)

        def stage_in(i, pp):
            return pltpu.make_async_copy(
                table_hbm.at[pl.ds(stage_row0(i), cpiece), pl.ds(c0, w_cols)],
                outb_v.at[pp], gsems[pp])

        def stage_out(i, pp):
            return pltpu.make_async_copy(
                buf_v.at[pp].at[pl.ds(0, cpiece)],
                tblk.at[pl.ds(stage_row0(i), cpiece)], osems[pp])

        stage_in(0, 0).start()

        def piece_body(j, _):
            for pp in (0, 1):
                i = 2 * j + pp
                ppn = (pp + 1) % 2

                @pl.when(i + 1 < n_pieces)
                def _():
                    stage_in(i + 1, ppn).start()

                stage_in(i, pp).wait()

                @pl.when(i >= 2)
                def _():
                    stage_out(i - 2, pp).wait()

                def crow(r, _):
                    for h in range(n_groups):
                        a = outb_v[pp, r, pl.ds(h * 2 * LANES, LANES)]
                        bq = outb_v[pp, r, pl.ds(h * 2 * LANES + LANES, LANES)]
                        buf_v[pp, r, pl.ds(h * 2 * LANES, 2 * LANES)] = (
                            plsc.pack(a, bq,
                                      format=plsc.PackFormat.INTERLEAVED))
                    return 0

                lax.fori_loop(0, cpiece, crow, 0)
                stage_out(i, pp).start()
            return 0

        lax.fori_loop(0, n_pieces // 2, piece_body, 0)
        for pp in (0, 1):
            stage_out(n_pieces - 2 + pp, pp).wait()
        plsc.subcore_barrier()

        def row_base(k):
            return jnp.minimum(sid * tile_rows + k * chunk, b - chunk)

        def idx_cps(k, q):
            rb = row_base(k)
            return (
                pltpu.make_async_copy(
                    neighs_hbm.at[pl.ds(rb * n_neigh, ngr)],
                    idx_v.at[q, pl.ds(0, ngr)], isems[q]),
                pltpu.make_async_copy(
                    nodes_hbm.at[pl.ds(rb, chunk)],
                    idx_v.at[q, pl.ds(ngr, chunk)], isems[q]),
            )

        def gather_cps(q, p, sem):
            return (
                pltpu.make_async_copy(
                    tblk.at[idx_v.at[q, pl.ds(0, ngr)]],
                    buf_v.at[p].at[pl.ds(0, ngr)], sem),
                pltpu.make_async_copy(
                    tblk.at[idx_v.at[q, pl.ds(ngr, chunk)]],
                    buf_v.at[p].at[pl.ds(ngr, chunk)], sem),
            )

        def start(cps):
            for cp in cps:
                cp.start()

        def wait(cps):
            for cp in cps:
                cp.wait()

        def out_slice(k):
            return out_hbm.at[pl.ds(row_base(k), chunk), pl.ds(c0, w_cols)]

        start(idx_cps(0, 0))
        wait(idx_cps(0, 0))
        start(gather_cps(0, 0, gsems[0]))
        start(idx_cps(1, 1))

        def six_body(i, _):
            for u in range(6):
                k = 6 * i + u
                p = u % 2
                pn = (p + 1) % 2
                q1 = (u + 1) % 3
                q2 = (u + 2) % 3

                @pl.when(k + 2 < n_chunks_pt)
                def _():
                    start(idx_cps(k + 2, q2))

                @pl.when(k + 1 < n_chunks_pt)
                def _():
                    wait(idx_cps(k + 1, q1))
                    start(gather_cps(q1, pn, gsems[pn]))

                wait(gather_cps(u % 3, p, gsems[p]))

                @pl.when(k >= 2)
                def _():
                    pltpu.make_async_copy(
                        outb_v.at[p].at[pl.ds(0, chunk)],
                        out_slice(k - 2), osems[p]).wait()

                def rbody(r, _):
                    rn = r * n_neigh
                    for h in range(n_groups):
                        col = pl.ds(h * 2 * LANES, 2 * LANES)
                        acc = buf_v[p, ngr + r, col]
                        for s in range(n_neigh):
                            acc = acc + buf_v[p, rn + s, col]
                        lo, hi = plsc.unpack(
                            acc, format=plsc.PackFormat.INTERLEAVED)
                        outb_v[p, r, pl.ds(h * 2 * LANES, LANES)] = (
                            lo * inv_count)
                        outb_v[p, r, pl.ds(h * 2 * LANES + LANES, LANES)] = (
                            hi * inv_count)
                    return 0

                lax.fori_loop(0, chunk, rbody, 0)
                pltpu.async_copy(outb_v.at[p].at[pl.ds(0, chunk)],
                                 out_slice(k), osems[p])
            return 0

        lax.fori_loop(0, n_chunks_pt // 6, six_body, 0)
        for p in (0, 1):
            k = n_chunks_pt - 2 + p
            pltpu.make_async_copy(
                outb_v.at[p].at[pl.ds(0, chunk)],
                out_slice(k), osems[p]).wait()

    return agg


def kernel(nodes, to_neighs, feature_table, num_sample):
    b = nodes.shape[0]
    n_neigh = to_neighs.shape[1]
    n_nodes, d_feat = feature_table.shape
    n_tiles = 16
    chunk = 32
    w_cols = d_feat // 2
    inv_count = 1.0 / float(n_neigh + 1)

    tile_rows = -(-b // n_tiles)
    tile_rows += (-tile_rows) % 8
    n_chunks_pt = -(-tile_rows // chunk)
    n_chunks_pt += (-n_chunks_pt) % 6

    agg = _build_sc_agg(n_nodes, d_feat, b, n_neigh, chunk, n_chunks_pt,
                        tile_rows, n_tiles, w_cols, inv_count)
    return agg(nodes, to_neighs.reshape(-1), feature_table)
